# baseline (speedup 1.0000x reference)
import jax
import jax.numpy as jnp
from jax import lax
from jax.experimental import pallas as pl
from jax.experimental.pallas import tpu as pltpu
from jax.experimental.pallas import tpu_sc as plsc

VOCAB = 1_000_000
DIM = 32
B, L = 16384, 50
STRIDE = 16
NSAMP = VOCAB // STRIDE
NPAD = 62504
PAD_VAL = 2**31 - 1

NC, NS = 2, 16
NW = NC * NS
ROWS_PER_W = B // NW
RCHUNK = 16
CIDS = RCHUNK * L
NCHUNK = ROWS_PER_W // RCHUNK


def _body(raw_hbm, keys2d_hbm, samp_hbm, table_hbm, out_hbm,
          samp_v, x_v, r_v, qrow_v, rows_v, emb_v, p_v, sem):
    wid = lax.axis_index("s") * NC + lax.axis_index("c")
    pltpu.sync_copy(samp_hbm, samp_v)

    lane = lax.broadcasted_iota(jnp.int32, (16,), 0)

    def chunk_body(k, carry):
        row_lo = wid * ROWS_PER_W + k * RCHUNK
        pltpu.sync_copy(raw_hbm.at[pl.ds(row_lo, RCHUNK)], x_v)

        for g in range(CIDS // 16):
            j = lane + g * 16
            xr = j // L
            xc = j - xr * L
            x = plsc.load_gather(x_v, [xr, xc])
            q = jnp.zeros((16,), jnp.int32)
            ln = NSAMP
            while ln > 1:
                half = ln >> 1
                probe = plsc.load_gather(samp_v, [q + (half - 1)])
                q = jnp.where(probe < x, q + half, q)
                ln -= half
            probe = plsc.load_gather(samp_v, [q])
            q = jnp.where(probe < x, q + 1, q)
            plsc.store_scatter(r_v, [xr, xc], jnp.maximum(q - 1, 0))

        cps = [pltpu.async_copy(keys2d_hbm.at[r_v.at[i]], qrow_v.at[i], sem)
               for i in range(RCHUNK)]
        for cp in cps:
            cp.wait()

        for g in range(CIDS // 16):
            j = lane + g * 16
            xr = j // L
            xc = j - xr * L
            x = plsc.load_gather(x_v, [xr, xc])
            r = plsc.load_gather(r_v, [xr, xc])
            c = jnp.zeros((16,), jnp.int32)
            for s in (8, 4, 2, 1):
                probe = plsc.load_gather(qrow_v, [xr, xc, c + (s - 1)])
                c = jnp.where(probe < x, c + s, c)
            probe = plsc.load_gather(qrow_v, [xr, xc, c])
            c = jnp.where(probe < x, c + 1, c)
            idx = r * STRIDE + c
            keyval_in = plsc.load_gather(qrow_v, [xr, xc, jnp.minimum(c, 15)])
            keyval_out = plsc.load_gather(samp_v, [r + 1])
            keyval = jnp.where(c < STRIDE, keyval_in, keyval_out)
            valid = (idx < VOCAB) & (keyval == x)
            plsc.store_scatter(rows_v, [xr, xc], jnp.where(valid, idx + 1, 0))

        cps = [pltpu.async_copy(table_hbm.at[rows_v.at[i]], emb_v.at[i], sem)
               for i in range(RCHUNK)]
        for cp in cps:
            cp.wait()

        def tr_body(l, carry2):
            lv = jnp.full((16,), l, jnp.int32)
            for d in range(DIM):
                v = plsc.load_gather(emb_v, [lane, lv, jnp.full((16,), d, jnp.int32)])
                plsc.store_scatter(p_v, [lv, jnp.full((16,), d, jnp.int32), lane], v)
            return carry2
        lax.fori_loop(0, L, tr_body, 0)
        pltpu.sync_copy(p_v, out_hbm.at[:, :, pl.ds(row_lo, RCHUNK)])
        return carry

    lax.fori_loop(0, NCHUNK, chunk_body, 0)


@jax.jit
def _lookup(raw_idx, keys2d, samp, table):
    mesh = plsc.VectorSubcoreMesh(core_axis_name="c", subcore_axis_name="s",
                                  num_cores=NC, num_subcores=NS)
    f = pl.kernel(
        _body,
        out_type=jax.ShapeDtypeStruct((L, DIM, B), jnp.float32),
        mesh=mesh,
        compiler_params=pltpu.CompilerParams(needs_layout_passes=False,
                                             use_tc_tiling_on_sc=False),
        scratch_types=[
            pltpu.VMEM((NPAD,), jnp.int32),
            pltpu.VMEM((RCHUNK, L), jnp.int32),
            pltpu.VMEM((RCHUNK, L), jnp.int32),
            pltpu.VMEM((RCHUNK, L, STRIDE), jnp.int32),
            pltpu.VMEM((RCHUNK, L), jnp.int32),
            pltpu.VMEM((RCHUNK, L, DIM), jnp.float32),
            pltpu.VMEM((L, DIM, RCHUNK), jnp.float32),
            pltpu.SemaphoreType.DMA,
        ],
    )
    return f(raw_idx, keys2d, samp, table)


def kernel(raw_idx, keys, table):
    keys2d = keys.reshape(NSAMP, STRIDE)
    samp = jnp.full((NPAD,), PAD_VAL, jnp.int32).at[:NSAMP].set(keys2d[:, 0])
    out_p = _lookup(raw_idx, keys2d, samp, table)
    return jnp.transpose(out_p, (2, 0, 1))


# speedup vs baseline: 1.1739x; 1.1739x over previous
import jax
import jax.numpy as jnp
from jax import lax
from jax.experimental import pallas as pl
from jax.experimental.pallas import tpu as pltpu
from jax.experimental.pallas import tpu_sc as plsc

VOCAB = 1_000_000
DIM = 32
B, L = 16384, 50
STRIDE = 16
NSAMP = VOCAB // STRIDE
NPAD = 62504
PAD_VAL = 2**31 - 1

NC, NS = 2, 16
NW = NC * NS
ROWS_PER_W = B // NW
RCHUNK = 16
CIDS = RCHUNK * L
NCHUNK = ROWS_PER_W // RCHUNK


def _body(raw_hbm, keys2d_hbm, samp_hbm, table_hbm, out_hbm,
          samp_v, x_v, r_v, qrow_v, rows_v, emb_v, p_v, sem):
    wid = lax.axis_index("s") * NC + lax.axis_index("c")
    pltpu.sync_copy(samp_hbm, samp_v)

    lane = lax.broadcasted_iota(jnp.int32, (16,), 0)

    def chunk_body(k, carry):
        row_lo = wid * ROWS_PER_W + k * RCHUNK
        pltpu.sync_copy(raw_hbm.at[pl.ds(row_lo, RCHUNK)], x_v)

        for g in range(CIDS // 16):
            j = lane + g * 16
            xr = j // L
            xc = j - xr * L
            x = plsc.load_gather(x_v, [xr, xc])
            q = jnp.zeros((16,), jnp.int32)
            ln = NSAMP
            while ln > 1:
                half = ln >> 1
                probe = plsc.load_gather(samp_v, [q + (half - 1)])
                q = jnp.where(probe < x, q + half, q)
                ln -= half
            probe = plsc.load_gather(samp_v, [q])
            q = jnp.where(probe < x, q + 1, q)
            plsc.store_scatter(r_v, [xr, xc], jnp.maximum(q - 1, 0))

        cps = [pltpu.async_copy(keys2d_hbm.at[r_v.at[i]], qrow_v.at[i], sem)
               for i in range(RCHUNK)]
        for cp in cps:
            cp.wait()

        for g in range(CIDS // 16):
            j = lane + g * 16
            xr = j // L
            xc = j - xr * L
            x = plsc.load_gather(x_v, [xr, xc])
            r = plsc.load_gather(r_v, [xr, xc])
            c = jnp.zeros((16,), jnp.int32)
            for s in (8, 4, 2, 1):
                probe = plsc.load_gather(qrow_v, [xr, xc, c + (s - 1)])
                c = jnp.where(probe < x, c + s, c)
            probe = plsc.load_gather(qrow_v, [xr, xc, c])
            c = jnp.where(probe < x, c + 1, c)
            idx = r * STRIDE + c
            keyval_in = plsc.load_gather(qrow_v, [xr, xc, jnp.minimum(c, 15)])
            keyval_out = plsc.load_gather(samp_v, [r + 1])
            keyval = jnp.where(c < STRIDE, keyval_in, keyval_out)
            valid = (idx < VOCAB) & (keyval == x)
            plsc.store_scatter(rows_v, [xr, xc], jnp.where(valid, idx + 1, 0))

        cps = [pltpu.async_copy(table_hbm.at[rows_v.at[i]], emb_v.at[i], sem)
               for i in range(RCHUNK)]
        for cp in cps:
            cp.wait()

        def tr_body(l, carry2):
            lv = jnp.full((16,), l, jnp.int32)
            vals = [plsc.load_gather(emb_v, [lane, lv, jnp.full((16,), d, jnp.int32)])
                    for d in range(DIM)]
            for d in range(DIM):
                p_v[l, d, :] = vals[d]
            return carry2
        lax.fori_loop(0, L, tr_body, 0)
        pltpu.sync_copy(p_v, out_hbm.at[:, :, pl.ds(row_lo, RCHUNK)])
        return carry

    lax.fori_loop(0, NCHUNK, chunk_body, 0)


@jax.jit
def _lookup(raw_idx, keys2d, samp, table):
    mesh = plsc.VectorSubcoreMesh(core_axis_name="c", subcore_axis_name="s",
                                  num_cores=NC, num_subcores=NS)
    f = pl.kernel(
        _body,
        out_type=jax.ShapeDtypeStruct((L, DIM, B), jnp.float32),
        mesh=mesh,
        compiler_params=pltpu.CompilerParams(needs_layout_passes=False,
                                             use_tc_tiling_on_sc=False),
        scratch_types=[
            pltpu.VMEM((NPAD,), jnp.int32),
            pltpu.VMEM((RCHUNK, L), jnp.int32),
            pltpu.VMEM((RCHUNK, L), jnp.int32),
            pltpu.VMEM((RCHUNK, L, STRIDE), jnp.int32),
            pltpu.VMEM((RCHUNK, L), jnp.int32),
            pltpu.VMEM((RCHUNK, L, DIM), jnp.float32),
            pltpu.VMEM((L, DIM, RCHUNK), jnp.float32),
            pltpu.SemaphoreType.DMA,
        ],
    )
    return f(raw_idx, keys2d, samp, table)


def kernel(raw_idx, keys, table):
    keys2d = keys.reshape(NSAMP, STRIDE)
    samp = jnp.full((NPAD,), PAD_VAL, jnp.int32).at[:NSAMP].set(keys2d[:, 0])
    out_p = _lookup(raw_idx, keys2d, samp, table)
    return jnp.transpose(out_p, (2, 0, 1))
